# baseline (speedup 1.0000x reference)
import jax
import jax.numpy as jnp
from jax import lax
from jax.experimental import pallas as pl
from jax.experimental.pallas import tpu as pltpu
from jax.experimental.pallas import tpu_sc as plsc

VOCAB = 1000000
D_MODEL = 64
BATCH = 4096
SEQ = 200

NC = 2
NS = 16
NW = NC * NS

IDX_BLK = 128
N_BLOCKS = BATCH * SEQ // IDX_BLK
BLOCKS_PER_W = N_BLOCKS // NW
K = 4


def _emb_body(tok_hbm, w_hbm, out_hbm, idx_v, rows_v, sem):
    wid = lax.axis_index("s") * NC + lax.axis_index("c")
    base = wid * BLOCKS_PER_W

    @pl.loop(0, BLOCKS_PER_W, step=K)
    def _(c):
        b0 = base + c
        pltpu.sync_copy(tok_hbm.at[pl.ds(b0, K)], idx_v)
        handles = [
            pltpu.async_copy(w_hbm.at[idx_v.at[j]], rows_v.at[j], sem)
            for j in range(K)
        ]
        for h in handles:
            h.wait()
        pltpu.sync_copy(rows_v, out_hbm.at[pl.ds(b0, K)])


@jax.jit
def _emb_lookup(tok2d, weight):
    mesh = plsc.VectorSubcoreMesh(core_axis_name="c", subcore_axis_name="s")
    run = pl.kernel(
        _emb_body,
        out_type=jax.ShapeDtypeStruct((N_BLOCKS, IDX_BLK, D_MODEL), jnp.float32),
        mesh=mesh,
        scratch_types=[
            pltpu.VMEM((K, IDX_BLK), jnp.int32),
            pltpu.VMEM((K, IDX_BLK, D_MODEL), jnp.float32),
            pltpu.SemaphoreType.DMA,
        ],
        compiler_params=pltpu.CompilerParams(use_tc_tiling_on_sc=False),
    )
    return run(tok2d, weight)


def kernel(token_ids, weight):
    tok2d = token_ids.reshape(N_BLOCKS, IDX_BLK).astype(jnp.int32)
    out = _emb_lookup(tok2d, weight)
    return out.reshape(BATCH, SEQ, D_MODEL)


# speedup vs baseline: 1.0421x; 1.0421x over previous
import jax
import jax.numpy as jnp
from jax import lax
from jax.experimental import pallas as pl
from jax.experimental.pallas import tpu as pltpu
from jax.experimental.pallas import tpu_sc as plsc

VOCAB = 1000000
D_MODEL = 64
BATCH = 4096
SEQ = 200

NC = 2
NS = 16
NW = NC * NS

IDX_BLK = 128
N_BLOCKS = BATCH * SEQ // IDX_BLK
BLOCKS_PER_W = N_BLOCKS // NW
K = 5
NBUF = 2
NCHUNK = BLOCKS_PER_W // K


def _emb_body(tok_hbm, w_hbm, out_hbm, idx_v, rows_v, gsem):
    wid = lax.axis_index("s") * NC + lax.axis_index("c")
    base = wid * BLOCKS_PER_W

    pltpu.sync_copy(tok_hbm.at[pl.ds(base, BLOCKS_PER_W)], idx_v)

    def fire(c, b):
        for j in range(K):
            pltpu.async_copy(w_hbm.at[idx_v.at[c * K + j]], rows_v.at[b, j], gsem)

    def drain(b):
        for j in range(K):
            pltpu.make_async_copy(w_hbm.at[idx_v.at[j]], rows_v.at[b, j], gsem).wait()

    for b in range(NBUF):
        fire(b, b)

    @pl.loop(0, NCHUNK - NBUF, step=NBUF)
    def _(c0):
        for b in range(NBUF):
            c = c0 + b
            drain(b)
            pltpu.sync_copy(rows_v.at[b], out_hbm.at[pl.ds(base + c * K, K)])
            fire(c + NBUF, b)

    for b in range(NBUF):
        c = NCHUNK - NBUF + b
        drain(b)
        pltpu.sync_copy(rows_v.at[b], out_hbm.at[pl.ds(base + c * K, K)])


@jax.jit
def _emb_lookup(tok2d, weight):
    mesh = plsc.VectorSubcoreMesh(core_axis_name="c", subcore_axis_name="s")
    run = pl.kernel(
        _emb_body,
        out_type=jax.ShapeDtypeStruct((N_BLOCKS, IDX_BLK, D_MODEL), jnp.float32),
        mesh=mesh,
        scratch_types=[
            pltpu.VMEM((BLOCKS_PER_W, IDX_BLK), jnp.int32),
            pltpu.VMEM((NBUF, K, IDX_BLK, D_MODEL), jnp.float32),
            pltpu.SemaphoreType.DMA,
        ],
        compiler_params=pltpu.CompilerParams(use_tc_tiling_on_sc=False),
    )
    return run(tok2d, weight)


def kernel(token_ids, weight):
    tok2d = token_ids.reshape(N_BLOCKS, IDX_BLK).astype(jnp.int32)
    out = _emb_lookup(tok2d, weight)
    return out.reshape(BATCH, SEQ, D_MODEL)
